# baseline (speedup 1.0000x reference)
import functools

import jax
import jax.numpy as jnp
from jax import lax
from jax.experimental import pallas as pl
from jax.experimental.pallas import tpu as pltpu
from jax.experimental.pallas import tpu_sc as plsc

_NUM_WORKERS = 32
_CH = 256
_NBUF = 4


def _make_gather(B, D):
    b_per_w = B // _NUM_WORKERS
    n_chunks = b_per_w // _CH
    assert n_chunks % _NBUF == 0 and n_chunks > _NBUF
    mesh = plsc.VectorSubcoreMesh(core_axis_name="c", subcore_axis_name="s")

    @functools.partial(
        pl.kernel,
        mesh=mesh,
        out_type=jax.ShapeDtypeStruct((B, D), jnp.float32),
        scratch_types=[
            pltpu.VMEM((b_per_w,), jnp.int32),
            pltpu.VMEM((_NBUF, _CH, D), jnp.float32),
        ]
        + [pltpu.SemaphoreType.DMA] * (2 * _NBUF),
        compiler_params=pltpu.CompilerParams(use_tc_tiling_on_sc=False),
    )
    def k(idx_hbm, table_hbm, out_hbm, idx_all, rows, *sems):
        gsems = sems[:_NBUF]
        wsems = sems[_NBUF:]
        wid = lax.axis_index("s") * 2 + lax.axis_index("c")
        base = wid * b_per_w
        pltpu.sync_copy(idx_hbm.at[pl.ds(base, b_per_w)], idx_all)

        def issue_gather(g, b):
            pltpu.async_copy(
                table_hbm.at[idx_all.at[pl.ds(g * _CH, _CH)]],
                rows.at[b],
                gsems[b],
            )

        def wait_gather(b):
            pltpu.make_async_copy(
                table_hbm.at[pl.ds(0, _CH)], rows.at[b], gsems[b]
            ).wait()

        def issue_writeback(g, b):
            pltpu.async_copy(
                rows.at[b], out_hbm.at[pl.ds(base + g * _CH, _CH)], wsems[b]
            )

        def wait_writeback(b):
            pltpu.make_async_copy(
                rows.at[b], out_hbm.at[pl.ds(base, _CH)], wsems[b]
            ).wait()

        for b in range(_NBUF - 1):
            issue_gather(b, b)

        def outer(i, carry):
            g0 = i * _NBUF
            for b in range(_NBUF):
                g = g0 + b
                wait_gather(b)
                issue_writeback(g, b)

                nb = (b + _NBUF - 1) % _NBUF

                @pl.when(g >= 1)
                def _():
                    wait_writeback(nb)

                @pl.when(g + (_NBUF - 1) < n_chunks)
                def _():
                    issue_gather(g + (_NBUF - 1), nb)

            return carry

        lax.fori_loop(0, n_chunks // _NBUF, outer, 0)
        wait_writeback((n_chunks - 1) % _NBUF)

    return k


def kernel(token_ids, weight):
    D = weight.shape[1]
    flat = token_ids.reshape(-1).astype(jnp.int32)
    out = _make_gather(flat.shape[0], D)(flat, weight)
    return out.reshape(*token_ids.shape, D)


# speedup vs baseline: 1.0048x; 1.0048x over previous
import functools

import jax
import jax.numpy as jnp
from jax import lax
from jax.experimental import pallas as pl
from jax.experimental.pallas import tpu as pltpu
from jax.experimental.pallas import tpu_sc as plsc

_NUM_WORKERS = 32
_CH = 256
_NBUF = 4


def _make_gather(B, D):
    b_per_w = B // _NUM_WORKERS
    n_chunks = b_per_w // _CH
    assert n_chunks % _NBUF == 0 and n_chunks > _NBUF
    mesh = plsc.VectorSubcoreMesh(core_axis_name="c", subcore_axis_name="s")

    @functools.partial(
        pl.kernel,
        mesh=mesh,
        out_type=jax.ShapeDtypeStruct((B, D), jnp.float32),
        scratch_types=[
            pltpu.VMEM((b_per_w,), jnp.int32),
            pltpu.VMEM((_NBUF, _CH, D), jnp.float32),
        ]
        + [pltpu.SemaphoreType.DMA] * (2 * _NBUF),
        compiler_params=pltpu.CompilerParams(use_tc_tiling_on_sc=False),
    )
    def k(idx_hbm, table_hbm, out_hbm, idx_all, rows, *sems):
        gsems = sems[:_NBUF]
        wsems = sems[_NBUF:]
        wid = lax.axis_index("s") * 2 + lax.axis_index("c")
        base = wid * b_per_w
        pltpu.sync_copy(idx_hbm.at[pl.ds(base, b_per_w)], idx_all)

        def issue_gather(g, b):
            for j in range(_CH // 16):
                iv = idx_all[pl.ds(g * _CH + j * 16, 16)]
                pltpu.async_copy(
                    table_hbm.at[iv],
                    rows.at[b].at[pl.ds(j * 16, 16)],
                    gsems[b],
                )

        def wait_gather(b):
            pltpu.make_async_copy(
                table_hbm.at[pl.ds(0, _CH)], rows.at[b], gsems[b]
            ).wait()

        def issue_writeback(g, b):
            pltpu.async_copy(
                rows.at[b], out_hbm.at[pl.ds(base + g * _CH, _CH)], wsems[b]
            )

        def wait_writeback(b):
            pltpu.make_async_copy(
                rows.at[b], out_hbm.at[pl.ds(base, _CH)], wsems[b]
            ).wait()

        for b in range(_NBUF - 1):
            issue_gather(b, b)

        def outer(i, carry):
            g0 = i * _NBUF
            for b in range(_NBUF):
                g = g0 + b
                wait_gather(b)
                issue_writeback(g, b)

                nb = (b + _NBUF - 1) % _NBUF

                @pl.when(g >= 1)
                def _():
                    wait_writeback(nb)

                @pl.when(g + (_NBUF - 1) < n_chunks)
                def _():
                    issue_gather(g + (_NBUF - 1), nb)

            return carry

        lax.fori_loop(0, n_chunks // _NBUF, outer, 0)
        wait_writeback((n_chunks - 1) % _NBUF)

    return k


def kernel(token_ids, weight):
    D = weight.shape[1]
    flat = token_ids.reshape(-1).astype(jnp.int32)
    out = _make_gather(flat.shape[0], D)(flat, weight)
    return out.reshape(*token_ids.shape, D)
